# baseline (speedup 1.0000x reference)
import functools

import jax
import jax.numpy as jnp
import numpy as np
from jax.experimental import pallas as pl
from jax.experimental.pallas import tpu as pltpu

HW = 80
K = 32
EPS = 1e-07
DELTA2 = 9.0
T = 400.0
_PREC = jax.lax.Precision.DEFAULT


def _bdot(a, b):
    return jnp.dot(a.astype(jnp.bfloat16), b.astype(jnp.bfloat16),
                   preferred_element_type=jnp.float32)


def _resize_matrix(n_in: int, n_out: int) -> np.ndarray:
    scale = n_in / n_out
    src = (np.arange(n_out) + 0.5) * scale - 0.5
    i0 = np.floor(src).astype(np.int64)
    t = (src - i0).astype(np.float64)
    R = np.zeros((n_out, n_in), np.float64)
    for i in range(n_out):
        for idx, w in ((i0[i], 1.0 - t[i]), (i0[i] + 1, t[i])):
            if 0 <= idx < n_in:
                R[i, idx] += w
        R[i] /= R[i].sum()
    return R.astype(np.float32)


def _jax_resize_matrix(n_in: int, n_out: int) -> np.ndarray:
    with jax.default_device(jax.devices("cpu")[0]):
        eye = np.eye(n_in, dtype=np.float32)
        r = jax.image.resize(eye, (n_out, n_in), method="bilinear")
        return np.asarray(jax.device_get(r), dtype=np.float32)


try:
    _RESIZE = _jax_resize_matrix(HW - 6, HW)
except Exception:
    _RESIZE = _resize_matrix(HW - 6, HW)


def _encode_values(x_chw, w1, b1, w2, b2, w3, b3, R, Rt):
    x = jnp.transpose(x_chw, (2, 1, 0)) - 0.33

    def conv(inp, w, b, n_in, n_out, side):
        acc = jnp.zeros((side * side, n_out), jnp.float32) + b
        for di in range(3):
            for dj in range(3):
                patch = inp[di:di + side, dj:dj + side, :].reshape(side * side, n_in)
                acc = acc + _bdot(patch, w[di, dj])
        return acc.reshape(side, side, n_out)

    x2 = jax.nn.relu(conv(x, w1, b1, 3, 20, HW - 2))
    x3 = jax.nn.relu(conv(x2, w2, b2, 20, 30, HW - 4))
    dense = conv(x3, w3, b3, 30, K, HW - 6)

    dT = jnp.transpose(dense, (0, 2, 1))
    a1 = jnp.dot(dT.reshape(74 * K, 74), Rt,
                 preferred_element_type=jnp.float32,
                 precision=jax.lax.Precision.HIGHEST)
    a1 = a1.reshape(74, K, HW)
    planes = []
    for k in range(K):
        u = jnp.dot(R, a1[:, k, :],
                    preferred_element_type=jnp.float32,
                    precision=jax.lax.Precision.HIGHEST)
        z = T * u
        e = jnp.exp(z - jnp.max(z))
        planes.append(e / jnp.sum(e))
    map_wf = jnp.stack(planes, axis=0)
    map_cf = jnp.transpose(map_wf, (0, 2, 1))

    wi = jax.lax.broadcasted_iota(jnp.int32, (K, HW, HW), 2).astype(jnp.float32)
    hi = jax.lax.broadcasted_iota(jnp.int32, (K, HW, HW), 1).astype(jnp.float32)
    col = jnp.sum(map_cf * wi, axis=(1, 2))
    row = jnp.sum(map_cf * hi, axis=(1, 2))
    return map_cf, col, row


def _enc_first_kernel(x_ref, xp_ref, w1_ref, b1_ref, w2_ref, b2_ref,
                      w3_ref, b3_ref, R_ref, Rt_ref,
                      map_ref, col_ref, row_ref, ic_ref, ov_ref):
    map_cf, col, row = _encode_values(
        x_ref[0], w1_ref[...], b1_ref[...], w2_ref[...], b2_ref[...],
        w3_ref[...], b3_ref[...], R_ref[...], Rt_ref[...])
    ic = (jnp.sum(jnp.abs(xp_ref[0] - x_ref[0]), axis=0) > 0).astype(jnp.float32)
    ov = jnp.sum(map_cf * ic[None, :, :], axis=(1, 2))
    map_ref[...] = map_cf[None]
    col_ref[...] = col.reshape(1, 1, K)
    row_ref[...] = row.reshape(1, 1, K)
    ic_ref[...] = ic[None]
    ov_ref[...] = ov.reshape(1, 1, K)


def _enc_map_kernel(x_ref, w1_ref, b1_ref, w2_ref, b2_ref, w3_ref, b3_ref,
                    R_ref, Rt_ref, map_ref, col_ref, row_ref):
    map_cf, col, row = _encode_values(
        x_ref[0], w1_ref[...], b1_ref[...], w2_ref[...], b2_ref[...],
        w3_ref[...], b3_ref[...], R_ref[...], Rt_ref[...])
    map_ref[...] = map_cf[None]
    col_ref[...] = col.reshape(1, 1, K)
    row_ref[...] = row.reshape(1, 1, K)


def _enc_kp_kernel(x_ref, w1_ref, b1_ref, w2_ref, b2_ref, w3_ref, b3_ref,
                   R_ref, Rt_ref, col_ref, row_ref):
    _, col, row = _encode_values(
        x_ref[0], w1_ref[...], b1_ref[...], w2_ref[...], b2_ref[...],
        w3_ref[...], b3_ref[...], R_ref[...], Rt_ref[...])
    col_ref[...] = col.reshape(1, 1, K)
    row_ref[...] = row.reshape(1, 1, K)


def _loss_kernel(c1_ref, r1_ref, c1p_ref, r1p_ref, ov_ref, ic_ref, out_ref):
    c1 = c1_ref[:, 0, :]
    r1 = r1_ref[:, 0, :]
    c1p = c1p_ref[:, 0, :]
    r1p = r1p_ref[:, 0, :]
    ov = ov_ref[:, 0, :]
    b = c1.shape[0]

    valid = (jnp.mean(ic_ref[...], axis=(1, 2)) > 0).astype(jnp.float32)
    nvalid = jnp.maximum(jnp.sum(valid), 1.0)

    kcl = jnp.mean((c1 - c1p) ** 2 + (r1 - r1p) ** 2)
    scl = jnp.sum(valid[:, None] * (-jnp.log(EPS + ov))) / (nvalid * K)
    ssl = jnp.sum(valid * (-jnp.log(EPS + jnp.sum(ov, axis=1)))) / nvalid

    d2 = ((c1[:, :, None] - c1[:, None, :2]) ** 2
          + (r1[:, :, None] - r1[:, None, :2]) ** 2)
    ki = jax.lax.broadcasted_iota(jnp.int32, (K, 2), 0)
    ji = jax.lax.broadcasted_iota(jnp.int32, (K, 2), 1)
    mask = (ki != ji).astype(jnp.float32)
    kvl = jnp.sum(jnp.maximum(DELTA2 - d2, 0.0) * mask[None]) / (K * K * b)

    out_ref[...] = jnp.stack([kcl, scl, kvl, ssl]).reshape(1, 4)


def _full_spec(shape):
    n = len(shape)
    return pl.BlockSpec(shape, lambda *a: (0,) * n)


def kernel(first, first_prev, second, w1, b1, w2, b2, w3, b3):
    B = first.shape[0]
    w1r = jnp.transpose(w1, (3, 2, 1, 0))
    w2r = jnp.transpose(w2, (3, 2, 1, 0))
    w3r = jnp.transpose(w3, (3, 2, 1, 0))
    b1r = b1.reshape(1, -1)
    b2r = b2.reshape(1, -1)
    b3r = b3.reshape(1, -1)
    R = jnp.asarray(_RESIZE)
    Rt = jnp.asarray(np.ascontiguousarray(_RESIZE.T))

    f32 = jnp.float32
    img_spec = pl.BlockSpec((1, 3, HW, HW), lambda b: (b, 0, 0, 0))
    map_spec = pl.BlockSpec((1, K, HW, HW), lambda b: (b, 0, 0, 0))
    kp_spec = pl.BlockSpec((1, 1, K), lambda b: (b, 0, 0))
    ic_spec = pl.BlockSpec((1, HW, HW), lambda b: (b, 0, 0))
    w_specs = [_full_spec(a.shape) for a in (w1r, b1r, w2r, b2r, w3r, b3r, R, Rt)]
    params = pltpu.CompilerParams(dimension_semantics=("parallel",),
                                  vmem_limit_bytes=63 * 1024 * 1024)

    map1, col1, row1, ic, ov = pl.pallas_call(
        _enc_first_kernel,
        grid=(B,),
        in_specs=[img_spec, img_spec] + w_specs,
        out_specs=[map_spec, kp_spec, kp_spec, ic_spec, kp_spec],
        out_shape=[
            jax.ShapeDtypeStruct((B, K, HW, HW), f32),
            jax.ShapeDtypeStruct((B, 1, K), f32),
            jax.ShapeDtypeStruct((B, 1, K), f32),
            jax.ShapeDtypeStruct((B, HW, HW), f32),
            jax.ShapeDtypeStruct((B, 1, K), f32),
        ],
        compiler_params=params,
    )(first, first_prev, w1r, b1r, w2r, b2r, w3r, b3r, R, Rt)

    map2, col2, row2 = pl.pallas_call(
        _enc_map_kernel,
        grid=(B,),
        in_specs=[img_spec] + w_specs,
        out_specs=[map_spec, kp_spec, kp_spec],
        out_shape=[
            jax.ShapeDtypeStruct((B, K, HW, HW), f32),
            jax.ShapeDtypeStruct((B, 1, K), f32),
            jax.ShapeDtypeStruct((B, 1, K), f32),
        ],
        compiler_params=params,
    )(second, w1r, b1r, w2r, b2r, w3r, b3r, R, Rt)

    col1p, row1p = pl.pallas_call(
        _enc_kp_kernel,
        grid=(B,),
        in_specs=[img_spec] + w_specs,
        out_specs=[kp_spec, kp_spec],
        out_shape=[
            jax.ShapeDtypeStruct((B, 1, K), f32),
            jax.ShapeDtypeStruct((B, 1, K), f32),
        ],
        compiler_params=params,
    )(first_prev, w1r, b1r, w2r, b2r, w3r, b3r, R, Rt)

    losses2d = pl.pallas_call(
        _loss_kernel,
        in_specs=[_full_spec((B, 1, K))] * 4
        + [_full_spec((B, 1, K)), _full_spec((B, HW, HW))],
        out_specs=pl.BlockSpec((1, 4), lambda *a: (0, 0)),
        out_shape=jax.ShapeDtypeStruct((1, 4), f32),
    )(col1, row1, col1p, row1p, ov, ic)

    kp1 = jnp.stack([col1[:, 0, :], row1[:, 0, :]], axis=2)
    kp2 = jnp.stack([col2[:, 0, :], row2[:, 0, :]], axis=2)
    kp1_prev = jnp.stack([col1p[:, 0, :], row1p[:, 0, :]], axis=2)
    return (kp1, kp2, kp1_prev, map1, map2, ic, losses2d[0])
